# Initial kernel scaffold; baseline (speedup 1.0000x reference)
import jax
import jax.numpy as jnp
from jax.experimental import pallas as pl


def kernel(x_company, edge_index_company, W_self1, W_nbr1, b1, W_self2, W_nbr2, b2, head_W1, head_b1, head_W2, head_b2):
    raise NotImplementedError("write your pallas kernel here")


# speedup vs baseline: 7.7145x; 7.7145x over previous
import functools

import jax
import jax.numpy as jnp
from jax import lax
from jax.experimental import pallas as pl
from jax.experimental.pallas import tpu as pltpu
from jax.experimental.pallas import tpu_sc as plsc

_N = 10000
_E = 160000
_D = 256
_H = 256
_EMB = 64
_HID = 32

_NC = 2
_NS = 16
_EPT = _E // _NS
_CHUNK = 200
_NCH = _EPT // _CHUNK
_RPT = 624
_ZR = 104
_REM = _N - _NS * _RPT
_FQ = 64
_F1A = 80
_F1B = 64
_BM = 400


def _make_seg_sum(F):
    mesh = plsc.VectorSubcoreMesh(
        core_axis_name="c", subcore_axis_name="s",
        num_cores=_NC, num_subcores=_NS,
    )

    @functools.partial(
        pl.kernel,
        out_type=[
            jax.ShapeDtypeStruct((_N, F), jnp.float32),
            jax.ShapeDtypeStruct((_N, F), jnp.float32),
        ],
        mesh=mesh,
        scratch_types=[
            pltpu.VMEM((_NCH, _CHUNK), jnp.int32),
            pltpu.VMEM((_NCH, _CHUNK), jnp.int32),
            pltpu.VMEM((_CHUNK, F), jnp.float32),
            pltpu.VMEM((_CHUNK, F), jnp.float32),
            pltpu.VMEM_SHARED((_N, F), jnp.float32),
            pltpu.SemaphoreType.DMA,
            pltpu.SemaphoreType.DMA,
        ],
        compiler_params=pltpu.CompilerParams(use_tc_tiling_on_sc=False),
    )
    def seg(tbl0_hbm, tbl1_hbm, src_hbm, dst_hbm, out0_hbm, out1_hbm,
            srcv, dstv, buf0, buf1, acc, sem0, sem1):
        c = lax.axis_index("c")
        s = lax.axis_index("s")

        pltpu.sync_copy(src_hbm.at[s], srcv)
        pltpu.sync_copy(dst_hbm.at[s], dstv)

        zero16 = jnp.zeros((16,), jnp.float32)

        def zrow(r, carry):
            for j in range(F // 16):
                buf0[r, pl.ds(j * 16, 16)] = zero16
            return carry

        lax.fori_loop(0, _ZR, zrow, 0)
        for k in range(_RPT // _ZR):
            pltpu.sync_copy(
                buf0.at[pl.ds(0, _ZR)],
                acc.at[pl.ds(s * _RPT + k * _ZR, _ZR)],
            )

        @pl.when(s == _NS - 1)
        def _():
            pltpu.sync_copy(
                buf0.at[pl.ds(0, _REM)],
                acc.at[pl.ds(_NS * _RPT, _REM)],
            )

        plsc.subcore_barrier()

        def main_loop(tbl_hbm):
            def gather(k, b, sem):
                return pltpu.make_async_copy(tbl_hbm.at[srcv.at[k]], b, sem)

            gather(0, buf0, sem0).start()
            gather(1, buf1, sem1).start()

            def step(i, carry):
                k0 = 2 * i
                gather(k0, buf0, sem0).wait()
                pltpu.sync_copy(buf0, acc.at[dstv.at[k0]], add=True)

                @pl.when(k0 + 2 < _NCH)
                def _():
                    gather(k0 + 2, buf0, sem0).start()

                k1 = k0 + 1
                gather(k1, buf1, sem1).wait()
                pltpu.sync_copy(buf1, acc.at[dstv.at[k1]], add=True)

                @pl.when(k1 + 2 < _NCH)
                def _():
                    gather(k1 + 2, buf1, sem1).start()

                return carry

            lax.fori_loop(0, _NCH // 2, step, 0)

        @pl.when(c == 0)
        def _():
            main_loop(tbl0_hbm)

        @pl.when(c == 1)
        def _():
            main_loop(tbl1_hbm)

        plsc.subcore_barrier()

        def copy_out(out_hbm):
            for k in range(_RPT // _ZR):
                base = s * _RPT + k * _ZR
                pltpu.sync_copy(
                    acc.at[pl.ds(base, _ZR)],
                    out_hbm.at[pl.ds(base, _ZR)],
                )

            @pl.when(s == _NS - 1)
            def _():
                pltpu.sync_copy(
                    acc.at[pl.ds(_NS * _RPT, _REM)],
                    out_hbm.at[pl.ds(_NS * _RPT, _REM)],
                )

        @pl.when(c == 0)
        def _():
            copy_out(out0_hbm)

        @pl.when(c == 1)
        def _():
            copy_out(out1_hbm)

    return seg


_seg_cache = {}


def _seg_sum(F, tbl0, tbl1, src3, dst3):
    if F not in _seg_cache:
        _seg_cache[F] = _make_seg_sum(F)
    return _seg_cache[F](tbl0, tbl1, src3, dst3)


def _tc1_body(x_ref, w_ref, a0_ref, a1_ref, b0_ref, b1_ref):
    mm = jnp.dot(x_ref[...], w_ref[...], preferred_element_type=jnp.float32)
    ones = jnp.ones((_BM, _F1A - _FQ), jnp.float32)
    a0_ref[:, 0:_FQ] = mm[:, 0:_FQ]
    a0_ref[:, _FQ:_F1A] = ones
    a1_ref[:, 0:_FQ] = mm[:, _FQ:2 * _FQ]
    a1_ref[:, _FQ:_F1A] = ones
    b0_ref[...] = mm[:, 2 * _FQ:3 * _FQ]
    b1_ref[...] = mm[:, 3 * _FQ:4 * _FQ]


def _tc1(x, w_nbr1):
    spec_a = pl.BlockSpec((_BM, _F1A), lambda i: (i, 0))
    spec_b = pl.BlockSpec((_BM, _F1B), lambda i: (i, 0))
    sds = jax.ShapeDtypeStruct
    return pl.pallas_call(
        _tc1_body,
        grid=(_N // _BM,),
        in_specs=[
            pl.BlockSpec((_BM, _D), lambda i: (i, 0)),
            pl.BlockSpec((_D, _H), lambda i: (0, 0)),
        ],
        out_specs=[spec_a, spec_a, spec_b, spec_b],
        out_shape=[
            sds((_N, _F1A), jnp.float32), sds((_N, _F1A), jnp.float32),
            sds((_N, _F1B), jnp.float32), sds((_N, _F1B), jnp.float32),
        ],
    )(x, w_nbr1)


def _tc2_body(x_ref, a0_ref, a1_ref, b0_ref, b1_ref, ws1_ref, bias1_ref,
              wn2_ref, h1_ref, p2a_ref, p2b_ref, rdeg_ref):
    cnt = a0_ref[:, _FQ:_FQ + 1]
    deg = jnp.maximum(cnt, 1.0)
    rdeg = 1.0 / deg
    agg = jnp.concatenate(
        [a0_ref[:, 0:_FQ], a1_ref[:, 0:_FQ], b0_ref[...], b1_ref[...]],
        axis=1) * rdeg
    h1 = jax.nn.relu(
        jnp.dot(x_ref[...], ws1_ref[...], preferred_element_type=jnp.float32)
        + agg + bias1_ref[...]
    )
    h1_ref[...] = h1
    p2 = jnp.dot(h1, wn2_ref[...], preferred_element_type=jnp.float32)
    p2a_ref[...] = p2[:, 0:32]
    p2b_ref[...] = p2[:, 32:64]
    rdeg_ref[...] = jnp.broadcast_to(rdeg, (_BM, 8))


def _tc2(x, s1a0, s1a1, s1b0, s1b1, w_self1, b1, w_nbr2):
    sds = jax.ShapeDtypeStruct
    return pl.pallas_call(
        _tc2_body,
        grid=(_N // _BM,),
        in_specs=[
            pl.BlockSpec((_BM, _D), lambda i: (i, 0)),
            pl.BlockSpec((_BM, _F1A), lambda i: (i, 0)),
            pl.BlockSpec((_BM, _F1A), lambda i: (i, 0)),
            pl.BlockSpec((_BM, _F1B), lambda i: (i, 0)),
            pl.BlockSpec((_BM, _F1B), lambda i: (i, 0)),
            pl.BlockSpec((_D, _H), lambda i: (0, 0)),
            pl.BlockSpec((1, _H), lambda i: (0, 0)),
            pl.BlockSpec((_H, _EMB), lambda i: (0, 0)),
        ],
        out_specs=[
            pl.BlockSpec((_BM, _H), lambda i: (i, 0)),
            pl.BlockSpec((_BM, 32), lambda i: (i, 0)),
            pl.BlockSpec((_BM, 32), lambda i: (i, 0)),
            pl.BlockSpec((_BM, 8), lambda i: (i, 0)),
        ],
        out_shape=[
            sds((_N, _H), jnp.float32),
            sds((_N, 32), jnp.float32),
            sds((_N, 32), jnp.float32),
            sds((_N, 8), jnp.float32),
        ],
    )(x, s1a0, s1a1, s1b0, s1b1, w_self1, b1, w_nbr2)


def _tc3_body(h1_ref, s2a_ref, s2b_ref, rdeg_ref, ws2_ref, b2_ref, hw1_ref,
              hb1_ref, hw2_ref, hb2_ref, out_ref):
    rd = rdeg_ref[:, 0:1]
    agg2 = jnp.concatenate([s2a_ref[...], s2b_ref[...]], axis=1) * rd
    emb = jax.nn.relu(
        jnp.dot(h1_ref[...], ws2_ref[...], preferred_element_type=jnp.float32)
        + agg2 + b2_ref[...]
    )
    hid = jax.nn.relu(
        jnp.dot(emb, hw1_ref[...], preferred_element_type=jnp.float32)
        + hb1_ref[...]
    )
    out_ref[...] = jnp.sum(hid * hw2_ref[...], axis=1, keepdims=True) + hb2_ref[...]


def _tc3(h1, s2a, s2b, rdeg, w_self2, b2, hw1, hb1, hw2r, hb2):
    return pl.pallas_call(
        _tc3_body,
        grid=(_N // _BM,),
        in_specs=[
            pl.BlockSpec((_BM, _H), lambda i: (i, 0)),
            pl.BlockSpec((_BM, 32), lambda i: (i, 0)),
            pl.BlockSpec((_BM, 32), lambda i: (i, 0)),
            pl.BlockSpec((_BM, 8), lambda i: (i, 0)),
            pl.BlockSpec((_H, _EMB), lambda i: (0, 0)),
            pl.BlockSpec((1, _EMB), lambda i: (0, 0)),
            pl.BlockSpec((_EMB, _HID), lambda i: (0, 0)),
            pl.BlockSpec((1, _HID), lambda i: (0, 0)),
            pl.BlockSpec((1, _HID), lambda i: (0, 0)),
            pl.BlockSpec((1, 1), lambda i: (0, 0)),
        ],
        out_specs=pl.BlockSpec((_BM, 1), lambda i: (i, 0)),
        out_shape=jax.ShapeDtypeStruct((_N, 1), jnp.float32),
    )(h1, s2a, s2b, rdeg, w_self2, b2, hw1, hb1, hw2r, hb2)


def kernel(x_company, edge_index_company, W_self1, W_nbr1, b1, W_self2,
           W_nbr2, b2, head_W1, head_b1, head_W2, head_b2):
    src3 = edge_index_company[0].reshape(_NS, _NCH, _CHUNK)
    dst3 = edge_index_company[1].reshape(_NS, _NCH, _CHUNK)

    t1a0, t1a1, t1b0, t1b1 = _tc1(x_company, W_nbr1)
    s1a0, s1a1 = _seg_sum(_F1A, t1a0, t1a1, src3, dst3)
    s1b0, s1b1 = _seg_sum(_F1B, t1b0, t1b1, src3, dst3)
    h1, p2a, p2b, rdeg = _tc2(x_company, s1a0, s1a1, s1b0, s1b1, W_self1,
                              b1.reshape(1, _H), W_nbr2)
    s2a, s2b = _seg_sum(32, p2a, p2b, src3, dst3)
    out = _tc3(h1, s2a, s2b, rdeg, W_self2, b2.reshape(1, _EMB), head_W1,
               head_b1.reshape(1, _HID), head_W2.reshape(1, _HID),
               head_b2.reshape(1, 1))
    return out[:, 0]


# speedup vs baseline: 8.8588x; 1.1483x over previous
import functools

import jax
import jax.numpy as jnp
from jax import lax
from jax.experimental import pallas as pl
from jax.experimental.pallas import tpu as pltpu
from jax.experimental.pallas import tpu_sc as plsc

_N = 10000
_E = 160000
_D = 256
_H = 256
_EMB = 64
_HID = 32

_NC = 2
_NS = 16
_EPT = _E // _NS
_CHUNK = 250
_NCH = _EPT // _CHUNK
_NBUF = 4
_RPT = 624
_ZR = 104
_REM = _N - _NS * _RPT
_FQ = 64
_BM = 400


def _make_seg_sum(val):
    F = val
    mesh = plsc.VectorSubcoreMesh(
        core_axis_name="c", subcore_axis_name="s",
        num_cores=_NC, num_subcores=_NS,
    )

    @functools.partial(
        pl.kernel,
        out_type=jax.ShapeDtypeStruct((_N, 128), jnp.float32),
        mesh=mesh,
        scratch_types=[
            pltpu.VMEM((_NCH, _CHUNK), jnp.int32),
            pltpu.VMEM((_NCH, _CHUNK), jnp.int32),
        ] + [pltpu.VMEM((_CHUNK, F), jnp.float32)] * _NBUF + [
            pltpu.VMEM_SHARED((_N, F), jnp.float32),
        ] + [pltpu.SemaphoreType.DMA] * (2 * _NBUF),
        compiler_params=pltpu.CompilerParams(use_tc_tiling_on_sc=False),
    )
    def seg(tbl0_hbm, tbl1_hbm, edge_hbm, out_hbm, srcv, dstv, *rest):
        bufs = rest[:_NBUF]
        acc = rest[_NBUF]
        gsems = rest[_NBUF + 1:_NBUF + 1 + _NBUF]
        ssems = rest[_NBUF + 1 + _NBUF:]

        c = lax.axis_index("c")
        s = lax.axis_index("s")

        pltpu.sync_copy(edge_hbm.at[0, s], srcv)
        pltpu.sync_copy(edge_hbm.at[1, s], dstv)

        zero16 = jnp.zeros((16,), jnp.float32)
        buf0 = bufs[0]

        def zrow(r, carry):
            for j in range(F // 16):
                buf0[r, pl.ds(j * 16, 16)] = zero16
            return carry

        lax.fori_loop(0, _ZR, zrow, 0)
        for k in range(_RPT // _ZR):
            pltpu.sync_copy(
                buf0.at[pl.ds(0, _ZR)],
                acc.at[pl.ds(s * _RPT + k * _ZR, _ZR)],
            )

        @pl.when(s == _NS - 1)
        def _():
            pltpu.sync_copy(
                buf0.at[pl.ds(0, _REM)],
                acc.at[pl.ds(_NS * _RPT, _REM)],
            )

        plsc.subcore_barrier()

        def main_loop(tbl_hbm):
            def gather(k, j):
                return pltpu.make_async_copy(
                    tbl_hbm.at[srcv.at[k]], bufs[j], gsems[j])

            def scatter(k, j):
                return pltpu.make_async_copy(
                    bufs[j], acc.at[dstv.at[k]], ssems[j])

            gather(0, 0).start()
            gather(1, 1).start()

            def step(i, carry):
                for j in range(_NBUF):
                    k = _NBUF * i + j
                    gather(k, j).wait()
                    pltpu.async_copy(
                        bufs[j], acc.at[dstv.at[k]], ssems[j], add=True)
                    jn = (j + 2) % _NBUF

                    @pl.when(k + 2 < _NCH)
                    def _():
                        @pl.when(k >= 2)
                        def _():
                            scatter(k - 2, jn).wait()

                        gather(k + 2, jn).start()

                return carry

            lax.fori_loop(0, _NCH // _NBUF, step, 0)
            for j in range(_NBUF):
                scatter(_NCH - _NBUF + j, (_NCH - _NBUF + j) % _NBUF).wait()

        @pl.when(c == 0)
        def _():
            main_loop(tbl0_hbm)

        @pl.when(c == 1)
        def _():
            main_loop(tbl1_hbm)

        plsc.subcore_barrier()

        def copy_rows(base, nrows):
            pltpu.sync_copy(
                acc.at[pl.ds(base, nrows), pl.ds(0, val)],
                out_hbm.at[pl.ds(base, nrows), pl.ds(c * val, val)],
            )

        for k in range(_RPT // _ZR):
            copy_rows(s * _RPT + k * _ZR, _ZR)

        @pl.when(s == _NS - 1)
        def _():
            copy_rows(_NS * _RPT, _REM)

    return seg


def _make_deg():
    mesh = plsc.VectorSubcoreMesh(
        core_axis_name="c", subcore_axis_name="s",
        num_cores=_NC, num_subcores=_NS,
    )

    @functools.partial(
        pl.kernel,
        out_type=[
            jax.ShapeDtypeStruct((_N, 16), jnp.float32),
            jax.ShapeDtypeStruct((_N, 16), jnp.float32),
        ],
        mesh=mesh,
        scratch_types=[
            pltpu.VMEM((_NCH, _CHUNK), jnp.int32),
            pltpu.VMEM((_CHUNK, 16), jnp.float32),
            pltpu.VMEM_SHARED((_N, 16), jnp.float32),
        ],
        compiler_params=pltpu.CompilerParams(use_tc_tiling_on_sc=False),
    )
    def deg(edge_hbm, out0_hbm, out1_hbm, dstv, ones, acc):
        c = lax.axis_index("c")
        s = lax.axis_index("s")
        pltpu.sync_copy(edge_hbm.at[1, s], dstv)

        zero16 = jnp.zeros((16,), jnp.float32)

        def zrow(r, carry):
            ones[r, pl.ds(0, 16)] = zero16
            return carry

        lax.fori_loop(0, _ZR, zrow, 0)
        for k in range(_RPT // _ZR):
            pltpu.sync_copy(
                ones.at[pl.ds(0, _ZR)],
                acc.at[pl.ds(s * _RPT + k * _ZR, _ZR)],
            )

        @pl.when(s == _NS - 1)
        def _():
            pltpu.sync_copy(
                ones.at[pl.ds(0, _REM)],
                acc.at[pl.ds(_NS * _RPT, _REM)],
            )

        one16 = jnp.ones((16,), jnp.float32)

        def orow(r, carry):
            ones[r, pl.ds(0, 16)] = one16
            return carry

        lax.fori_loop(0, _CHUNK, orow, 0)
        plsc.subcore_barrier()

        half = _NCH // _NC

        def step(k, carry):
            pltpu.sync_copy(ones, acc.at[dstv.at[k]], add=True)
            return carry

        lax.fori_loop(c * half, (c + 1) * half, step, 0)
        plsc.subcore_barrier()

        def copy_out(out_hbm):
            for k in range(_RPT // _ZR):
                base = s * _RPT + k * _ZR
                pltpu.sync_copy(
                    acc.at[pl.ds(base, _ZR)],
                    out_hbm.at[pl.ds(base, _ZR)],
                )

            @pl.when(s == _NS - 1)
            def _():
                pltpu.sync_copy(
                    acc.at[pl.ds(_NS * _RPT, _REM)],
                    out_hbm.at[pl.ds(_NS * _RPT, _REM)],
                )

        @pl.when(c == 0)
        def _():
            copy_out(out0_hbm)

        @pl.when(c == 1)
        def _():
            copy_out(out1_hbm)

    return deg


_seg_cache = {}


def _seg_sum(val, tbl0, tbl1, edges):
    if val not in _seg_cache:
        _seg_cache[val] = _make_seg_sum(val)
    return _seg_cache[val](tbl0, tbl1, edges)


def _deg_sum(edges):
    if "deg" not in _seg_cache:
        _seg_cache["deg"] = _make_deg()
    return _seg_cache["deg"](edges)


def _tc1_body(x_ref, w_ref, b0_ref, b1_ref):
    mm = jnp.dot(x_ref[...], w_ref[...], preferred_element_type=jnp.float32)
    b0_ref[...] = mm[:, 0:_FQ]
    b1_ref[...] = mm[:, _FQ:2 * _FQ]


def _tc1(x, w_half):
    spec_b = pl.BlockSpec((_BM, _FQ), lambda i: (i, 0))
    sds = jax.ShapeDtypeStruct
    return pl.pallas_call(
        _tc1_body,
        grid=(_N // _BM,),
        in_specs=[
            pl.BlockSpec((_BM, _D), lambda i: (i, 0)),
            pl.BlockSpec((_D, 2 * _FQ), lambda i: (0, 0)),
        ],
        out_specs=[spec_b, spec_b],
        out_shape=[sds((_N, _FQ), jnp.float32)] * 2,
    )(x, w_half)


def _tc_self1_body(x_ref, w_ref, b_ref, out_ref):
    out_ref[...] = (
        jnp.dot(x_ref[...], w_ref[...], preferred_element_type=jnp.float32)
        + b_ref[...]
    )


def _tc_self1(x, w_self1, b1):
    return pl.pallas_call(
        _tc_self1_body,
        grid=(_N // _BM,),
        in_specs=[
            pl.BlockSpec((_BM, _D), lambda i: (i, 0)),
            pl.BlockSpec((_D, _H), lambda i: (0, 0)),
            pl.BlockSpec((1, _H), lambda i: (0, 0)),
        ],
        out_specs=pl.BlockSpec((_BM, _H), lambda i: (i, 0)),
        out_shape=jax.ShapeDtypeStruct((_N, _H), jnp.float32),
    )(x, w_self1, b1)


def _tc2_body(sself_ref, s1a_ref, s1b_ref, d0_ref, d1_ref, wn2_ref,
              h1_ref, p2a_ref, p2b_ref, rdeg_ref):
    deg = jnp.maximum(d0_ref[:, 0:1] + d1_ref[:, 0:1], 1.0)
    rdeg = 1.0 / deg
    agg = jnp.concatenate([s1a_ref[...], s1b_ref[...]], axis=1) * rdeg
    h1 = jax.nn.relu(sself_ref[...] + agg)
    h1_ref[...] = h1
    p2 = jnp.dot(h1, wn2_ref[...], preferred_element_type=jnp.float32)
    p2a_ref[...] = p2[:, 0:32]
    p2b_ref[...] = p2[:, 32:64]
    rdeg_ref[...] = jnp.broadcast_to(rdeg, (_BM, 8))


def _tc2(sself, s1a, s1b, d0, d1, w_nbr2):
    sds = jax.ShapeDtypeStruct
    return pl.pallas_call(
        _tc2_body,
        grid=(_N // _BM,),
        in_specs=[
            pl.BlockSpec((_BM, _H), lambda i: (i, 0)),
            pl.BlockSpec((_BM, 128), lambda i: (i, 0)),
            pl.BlockSpec((_BM, 128), lambda i: (i, 0)),
            pl.BlockSpec((_BM, 16), lambda i: (i, 0)),
            pl.BlockSpec((_BM, 16), lambda i: (i, 0)),
            pl.BlockSpec((_H, _EMB), lambda i: (0, 0)),
        ],
        out_specs=[
            pl.BlockSpec((_BM, _H), lambda i: (i, 0)),
            pl.BlockSpec((_BM, 32), lambda i: (i, 0)),
            pl.BlockSpec((_BM, 32), lambda i: (i, 0)),
            pl.BlockSpec((_BM, 8), lambda i: (i, 0)),
        ],
        out_shape=[
            sds((_N, _H), jnp.float32),
            sds((_N, 32), jnp.float32),
            sds((_N, 32), jnp.float32),
            sds((_N, 8), jnp.float32),
        ],
    )(sself, s1a, s1b, d0, d1, w_nbr2)


def _tc_self2_body(h1_ref, w_ref, b_ref, out_ref):
    out_ref[...] = (
        jnp.dot(h1_ref[...], w_ref[...], preferred_element_type=jnp.float32)
        + b_ref[...]
    )


def _tc_self2(h1, w_self2, b2):
    return pl.pallas_call(
        _tc_self2_body,
        grid=(_N // _BM,),
        in_specs=[
            pl.BlockSpec((_BM, _H), lambda i: (i, 0)),
            pl.BlockSpec((_H, _EMB), lambda i: (0, 0)),
            pl.BlockSpec((1, _EMB), lambda i: (0, 0)),
        ],
        out_specs=pl.BlockSpec((_BM, _EMB), lambda i: (i, 0)),
        out_shape=jax.ShapeDtypeStruct((_N, _EMB), jnp.float32),
    )(h1, w_self2, b2)


def _tc3_body(s2self_ref, s2_ref, rdeg_ref, hw1_ref, hb1_ref, hw2_ref,
              hb2_ref, out_ref):
    rd = rdeg_ref[:, 0:1]
    agg2 = s2_ref[:, 0:_EMB] * rd
    emb = jax.nn.relu(s2self_ref[...] + agg2)
    hid = jax.nn.relu(
        jnp.dot(emb, hw1_ref[...], preferred_element_type=jnp.float32)
        + hb1_ref[...]
    )
    out_ref[...] = jnp.sum(hid * hw2_ref[...], axis=1, keepdims=True) + hb2_ref[...]


def _tc3(s2self, s2, rdeg, hw1, hb1, hw2r, hb2):
    return pl.pallas_call(
        _tc3_body,
        grid=(_N // _BM,),
        in_specs=[
            pl.BlockSpec((_BM, _EMB), lambda i: (i, 0)),
            pl.BlockSpec((_BM, 128), lambda i: (i, 0)),
            pl.BlockSpec((_BM, 8), lambda i: (i, 0)),
            pl.BlockSpec((_EMB, _HID), lambda i: (0, 0)),
            pl.BlockSpec((1, _HID), lambda i: (0, 0)),
            pl.BlockSpec((1, _HID), lambda i: (0, 0)),
            pl.BlockSpec((1, 1), lambda i: (0, 0)),
        ],
        out_specs=pl.BlockSpec((_BM, 1), lambda i: (i, 0)),
        out_shape=jax.ShapeDtypeStruct((_N, 1), jnp.float32),
    )(s2self, s2, rdeg, hw1, hb1, hw2r, hb2)


def kernel(x_company, edge_index_company, W_self1, W_nbr1, b1, W_self2,
           W_nbr2, b2, head_W1, head_b1, head_W2, head_b2):
    edges = edge_index_company.reshape(2, _NS, _NCH, _CHUNK)

    d0, d1 = _deg_sum(edges)
    t1a0, t1a1 = _tc1(x_company, W_nbr1[:, 0:2 * _FQ])
    s1a = _seg_sum(_FQ, t1a0, t1a1, edges)
    t1b0, t1b1 = _tc1(x_company, W_nbr1[:, 2 * _FQ:4 * _FQ])
    s1b = _seg_sum(_FQ, t1b0, t1b1, edges)
    s1self = _tc_self1(x_company, W_self1, b1.reshape(1, _H))
    h1, p2a, p2b, rdeg = _tc2(s1self, s1a, s1b, d0, d1, W_nbr2)
    s2 = _seg_sum(32, p2a, p2b, edges)
    s2self = _tc_self2(h1, W_self2, b2.reshape(1, _EMB))
    out = _tc3(s2self, s2, rdeg, head_W1, head_b1.reshape(1, _HID),
               head_W2.reshape(1, _HID), head_b2.reshape(1, 1))
    return out[:, 0]


# speedup vs baseline: 9.2554x; 1.0448x over previous
import functools

import jax
import jax.numpy as jnp
from jax import lax
from jax.experimental import pallas as pl
from jax.experimental.pallas import tpu as pltpu
from jax.experimental.pallas import tpu_sc as plsc

_N = 10000
_E = 160000
_D = 256
_H = 256
_EMB = 64
_HID = 32

_NC = 2
_NS = 16
_EPT = _E // _NS
_CHUNK = 250
_NCH = _EPT // _CHUNK
_NBUF = 4
_RPT = 624
_ZR = 104
_REM = _N - _NS * _RPT
_FQ = 64
_BM = 400


def _make_seg_sum(val):
    F = val
    mesh = plsc.VectorSubcoreMesh(
        core_axis_name="c", subcore_axis_name="s",
        num_cores=_NC, num_subcores=_NS,
    )

    @functools.partial(
        pl.kernel,
        out_type=jax.ShapeDtypeStruct((_N, 128), jnp.bfloat16),
        mesh=mesh,
        scratch_types=[
            pltpu.VMEM((_NCH, _CHUNK), jnp.int32),
            pltpu.VMEM((_NCH, _CHUNK), jnp.int32),
        ] + [pltpu.VMEM((_CHUNK, F), jnp.bfloat16)] * _NBUF + [
            pltpu.VMEM_SHARED((_N, F), jnp.bfloat16),
        ] + [pltpu.SemaphoreType.DMA] * (2 * _NBUF),
        compiler_params=pltpu.CompilerParams(use_tc_tiling_on_sc=False),
    )
    def seg(tbl0_hbm, tbl1_hbm, edge_hbm, out_hbm, srcv, dstv, *rest):
        bufs = rest[:_NBUF]
        acc = rest[_NBUF]
        gsems = rest[_NBUF + 1:_NBUF + 1 + _NBUF]
        ssems = rest[_NBUF + 1 + _NBUF:]

        c = lax.axis_index("c")
        s = lax.axis_index("s")

        pltpu.sync_copy(edge_hbm.at[0, s], srcv)
        pltpu.sync_copy(edge_hbm.at[1, s], dstv)

        zero32 = jnp.zeros((32,), jnp.bfloat16)
        buf0 = bufs[0]

        def zrow(r, carry):
            for j in range(F // 32):
                buf0[r, pl.ds(j * 32, 32)] = zero32
            return carry

        lax.fori_loop(0, _ZR, zrow, 0)
        for k in range(_RPT // _ZR):
            pltpu.sync_copy(
                buf0.at[pl.ds(0, _ZR)],
                acc.at[pl.ds(s * _RPT + k * _ZR, _ZR)],
            )

        @pl.when(s == _NS - 1)
        def _():
            pltpu.sync_copy(
                buf0.at[pl.ds(0, _REM)],
                acc.at[pl.ds(_NS * _RPT, _REM)],
            )

        plsc.subcore_barrier()

        def main_loop(tbl_hbm):
            def gather(k, j):
                return pltpu.make_async_copy(
                    tbl_hbm.at[srcv.at[k]], bufs[j], gsems[j])

            def scatter(k, j):
                return pltpu.make_async_copy(
                    bufs[j], acc.at[dstv.at[k]], ssems[j])

            gather(0, 0).start()
            gather(1, 1).start()

            def step(i, carry):
                for j in range(_NBUF):
                    k = _NBUF * i + j
                    gather(k, j).wait()
                    pltpu.async_copy(
                        bufs[j], acc.at[dstv.at[k]], ssems[j], add=True)
                    jn = (j + 2) % _NBUF

                    @pl.when(k + 2 < _NCH)
                    def _():
                        @pl.when(k >= 2)
                        def _():
                            scatter(k - 2, jn).wait()

                        gather(k + 2, jn).start()

                return carry

            lax.fori_loop(0, _NCH // _NBUF, step, 0)
            for j in range(_NBUF):
                scatter(_NCH - _NBUF + j, (_NCH - _NBUF + j) % _NBUF).wait()

        @pl.when(c == 0)
        def _():
            main_loop(tbl0_hbm)

        @pl.when(c == 1)
        def _():
            main_loop(tbl1_hbm)

        plsc.subcore_barrier()

        def copy_rows(base, nrows):
            pltpu.sync_copy(
                acc.at[pl.ds(base, nrows), pl.ds(0, val)],
                out_hbm.at[pl.ds(base, nrows), pl.ds(c * val, val)],
            )

        for k in range(_RPT // _ZR):
            copy_rows(s * _RPT + k * _ZR, _ZR)

        @pl.when(s == _NS - 1)
        def _():
            copy_rows(_NS * _RPT, _REM)

    return seg


def _make_deg():
    mesh = plsc.VectorSubcoreMesh(
        core_axis_name="c", subcore_axis_name="s",
        num_cores=_NC, num_subcores=_NS,
    )

    @functools.partial(
        pl.kernel,
        out_type=[
            jax.ShapeDtypeStruct((_N, 16), jnp.float32),
            jax.ShapeDtypeStruct((_N, 16), jnp.float32),
        ],
        mesh=mesh,
        scratch_types=[
            pltpu.VMEM((_NCH, _CHUNK), jnp.int32),
            pltpu.VMEM((_CHUNK, 16), jnp.float32),
            pltpu.VMEM_SHARED((_N, 16), jnp.float32),
        ],
        compiler_params=pltpu.CompilerParams(use_tc_tiling_on_sc=False),
    )
    def deg(edge_hbm, out0_hbm, out1_hbm, dstv, ones, acc):
        c = lax.axis_index("c")
        s = lax.axis_index("s")
        pltpu.sync_copy(edge_hbm.at[1, s], dstv)

        zero16 = jnp.zeros((16,), jnp.float32)

        def zrow(r, carry):
            ones[r, pl.ds(0, 16)] = zero16
            return carry

        lax.fori_loop(0, _ZR, zrow, 0)
        for k in range(_RPT // _ZR):
            pltpu.sync_copy(
                ones.at[pl.ds(0, _ZR)],
                acc.at[pl.ds(s * _RPT + k * _ZR, _ZR)],
            )

        @pl.when(s == _NS - 1)
        def _():
            pltpu.sync_copy(
                ones.at[pl.ds(0, _REM)],
                acc.at[pl.ds(_NS * _RPT, _REM)],
            )

        one16 = jnp.ones((16,), jnp.float32)

        def orow(r, carry):
            ones[r, pl.ds(0, 16)] = one16
            return carry

        lax.fori_loop(0, _CHUNK, orow, 0)
        plsc.subcore_barrier()

        half = _NCH // _NC

        def step(k, carry):
            pltpu.sync_copy(ones, acc.at[dstv.at[k]], add=True)
            return carry

        lax.fori_loop(c * half, (c + 1) * half, step, 0)
        plsc.subcore_barrier()

        def copy_out(out_hbm):
            for k in range(_RPT // _ZR):
                base = s * _RPT + k * _ZR
                pltpu.sync_copy(
                    acc.at[pl.ds(base, _ZR)],
                    out_hbm.at[pl.ds(base, _ZR)],
                )

            @pl.when(s == _NS - 1)
            def _():
                pltpu.sync_copy(
                    acc.at[pl.ds(_NS * _RPT, _REM)],
                    out_hbm.at[pl.ds(_NS * _RPT, _REM)],
                )

        @pl.when(c == 0)
        def _():
            copy_out(out0_hbm)

        @pl.when(c == 1)
        def _():
            copy_out(out1_hbm)

    return deg


_seg_cache = {}


def _seg_sum(val, tbl0, tbl1, edges):
    if val not in _seg_cache:
        _seg_cache[val] = _make_seg_sum(val)
    return _seg_cache[val](tbl0, tbl1, edges)


def _deg_sum(edges):
    if "deg" not in _seg_cache:
        _seg_cache["deg"] = _make_deg()
    return _seg_cache["deg"](edges)


def _tc1_body(x_ref, w_ref, b0_ref, b1_ref):
    mm = jnp.dot(x_ref[...], w_ref[...], preferred_element_type=jnp.float32)
    mm = mm.astype(jnp.bfloat16)
    b0_ref[...] = mm[:, 0:_FQ]
    b1_ref[...] = mm[:, _FQ:2 * _FQ]


def _tc1(x, w_half):
    spec_b = pl.BlockSpec((_BM, _FQ), lambda i: (i, 0))
    sds = jax.ShapeDtypeStruct
    return pl.pallas_call(
        _tc1_body,
        grid=(_N // _BM,),
        in_specs=[
            pl.BlockSpec((_BM, _D), lambda i: (i, 0)),
            pl.BlockSpec((_D, 2 * _FQ), lambda i: (0, 0)),
        ],
        out_specs=[spec_b, spec_b],
        out_shape=[sds((_N, _FQ), jnp.bfloat16)] * 2,
    )(x, w_half)


def _tc_self1_body(x_ref, w_ref, b_ref, out_ref):
    out_ref[...] = (
        jnp.dot(x_ref[...], w_ref[...], preferred_element_type=jnp.float32)
        + b_ref[...]
    )


def _tc_self1(x, w_self1, b1):
    return pl.pallas_call(
        _tc_self1_body,
        grid=(_N // _BM,),
        in_specs=[
            pl.BlockSpec((_BM, _D), lambda i: (i, 0)),
            pl.BlockSpec((_D, _H), lambda i: (0, 0)),
            pl.BlockSpec((1, _H), lambda i: (0, 0)),
        ],
        out_specs=pl.BlockSpec((_BM, _H), lambda i: (i, 0)),
        out_shape=jax.ShapeDtypeStruct((_N, _H), jnp.float32),
    )(x, w_self1, b1)


def _tc2_body(sself_ref, s1a_ref, s1b_ref, d0_ref, d1_ref, wn2_ref,
              h1_ref, p2a_ref, p2b_ref, rdeg_ref):
    deg = jnp.maximum(d0_ref[:, 0:1] + d1_ref[:, 0:1], 1.0)
    rdeg = 1.0 / deg
    agg = jnp.concatenate(
        [s1a_ref[...], s1b_ref[...]], axis=1).astype(jnp.float32) * rdeg
    h1 = jax.nn.relu(sself_ref[...] + agg)
    h1_ref[...] = h1
    p2 = jnp.dot(h1, wn2_ref[...], preferred_element_type=jnp.float32)
    p2 = p2.astype(jnp.bfloat16)
    p2a_ref[...] = p2[:, 0:32]
    p2b_ref[...] = p2[:, 32:64]
    rdeg_ref[...] = jnp.broadcast_to(rdeg, (_BM, 8))


def _tc2(sself, s1a, s1b, d0, d1, w_nbr2):
    sds = jax.ShapeDtypeStruct
    return pl.pallas_call(
        _tc2_body,
        grid=(_N // _BM,),
        in_specs=[
            pl.BlockSpec((_BM, _H), lambda i: (i, 0)),
            pl.BlockSpec((_BM, 128), lambda i: (i, 0)),
            pl.BlockSpec((_BM, 128), lambda i: (i, 0)),
            pl.BlockSpec((_BM, 16), lambda i: (i, 0)),
            pl.BlockSpec((_BM, 16), lambda i: (i, 0)),
            pl.BlockSpec((_H, _EMB), lambda i: (0, 0)),
        ],
        out_specs=[
            pl.BlockSpec((_BM, _H), lambda i: (i, 0)),
            pl.BlockSpec((_BM, 32), lambda i: (i, 0)),
            pl.BlockSpec((_BM, 32), lambda i: (i, 0)),
            pl.BlockSpec((_BM, 8), lambda i: (i, 0)),
        ],
        out_shape=[
            sds((_N, _H), jnp.float32),
            sds((_N, 32), jnp.bfloat16),
            sds((_N, 32), jnp.bfloat16),
            sds((_N, 8), jnp.float32),
        ],
    )(sself, s1a, s1b, d0, d1, w_nbr2)


def _tc_self2_body(h1_ref, w_ref, b_ref, out_ref):
    out_ref[...] = (
        jnp.dot(h1_ref[...], w_ref[...], preferred_element_type=jnp.float32)
        + b_ref[...]
    )


def _tc_self2(h1, w_self2, b2):
    return pl.pallas_call(
        _tc_self2_body,
        grid=(_N // _BM,),
        in_specs=[
            pl.BlockSpec((_BM, _H), lambda i: (i, 0)),
            pl.BlockSpec((_H, _EMB), lambda i: (0, 0)),
            pl.BlockSpec((1, _EMB), lambda i: (0, 0)),
        ],
        out_specs=pl.BlockSpec((_BM, _EMB), lambda i: (i, 0)),
        out_shape=jax.ShapeDtypeStruct((_N, _EMB), jnp.float32),
    )(h1, w_self2, b2)


def _tc3_body(s2self_ref, s2_ref, rdeg_ref, hw1_ref, hb1_ref, hw2_ref,
              hb2_ref, out_ref):
    rd = rdeg_ref[:, 0:1]
    agg2 = s2_ref[:, 0:_EMB].astype(jnp.float32) * rd
    emb = jax.nn.relu(s2self_ref[...] + agg2)
    hid = jax.nn.relu(
        jnp.dot(emb, hw1_ref[...], preferred_element_type=jnp.float32)
        + hb1_ref[...]
    )
    out_ref[...] = jnp.sum(hid * hw2_ref[...], axis=1, keepdims=True) + hb2_ref[...]


def _tc3(s2self, s2, rdeg, hw1, hb1, hw2r, hb2):
    return pl.pallas_call(
        _tc3_body,
        grid=(_N // _BM,),
        in_specs=[
            pl.BlockSpec((_BM, _EMB), lambda i: (i, 0)),
            pl.BlockSpec((_BM, 128), lambda i: (i, 0)),
            pl.BlockSpec((_BM, 8), lambda i: (i, 0)),
            pl.BlockSpec((_EMB, _HID), lambda i: (0, 0)),
            pl.BlockSpec((1, _HID), lambda i: (0, 0)),
            pl.BlockSpec((1, _HID), lambda i: (0, 0)),
            pl.BlockSpec((1, 1), lambda i: (0, 0)),
        ],
        out_specs=pl.BlockSpec((_BM, 1), lambda i: (i, 0)),
        out_shape=jax.ShapeDtypeStruct((_N, 1), jnp.float32),
    )(s2self, s2, rdeg, hw1, hb1, hw2r, hb2)


def kernel(x_company, edge_index_company, W_self1, W_nbr1, b1, W_self2,
           W_nbr2, b2, head_W1, head_b1, head_W2, head_b2):
    edges = edge_index_company.reshape(2, _NS, _NCH, _CHUNK)

    d0, d1 = _deg_sum(edges)
    t1a0, t1a1 = _tc1(x_company, W_nbr1[:, 0:2 * _FQ])
    s1a = _seg_sum(_FQ, t1a0, t1a1, edges)
    t1b0, t1b1 = _tc1(x_company, W_nbr1[:, 2 * _FQ:4 * _FQ])
    s1b = _seg_sum(_FQ, t1b0, t1b1, edges)
    s1self = _tc_self1(x_company, W_self1, b1.reshape(1, _H))
    h1, p2a, p2b, rdeg = _tc2(s1self, s1a, s1b, d0, d1, W_nbr2)
    s2 = _seg_sum(32, p2a, p2b, edges)
    s2self = _tc_self2(h1, W_self2, b2.reshape(1, _EMB))
    out = _tc3(s2self, s2, rdeg, head_W1, head_b1.reshape(1, _HID),
               head_W2.reshape(1, _HID), head_b2.reshape(1, 1))
    return out[:, 0]
